# baseline (speedup 1.0000x reference)
import jax
import jax.numpy as jnp
from jax import lax
from jax.experimental import pallas as pl
from jax.experimental.pallas import tpu as pltpu
from jax.experimental.pallas import tpu_sc as plsc

D = 32
SEQ = 50
B = 16384
L = 16
NC, NS = 2, 16
NACT = 8
NW = NC * NACT
SENT_PER_W = B // NW
CHS = 64
NCHUNK = SENT_PER_W // CHS
TOK_PER_CHUNK = CHS * SEQ


def _body(ids_hbm, table_hbm, out_hbm, idx_v, rows_v, out_v, sem):
    sid = lax.axis_index("s")
    cid = lax.axis_index("c")
    wid = sid * NC + cid

    @pl.when(sid < NACT)
    def _():
        def chunk(c, carry):
            tok0 = wid * (SENT_PER_W * SEQ) + c * TOK_PER_CHUNK
            pltpu.sync_copy(ids_hbm.at[pl.ds(tok0, TOK_PER_CHUNK)], idx_v)
            pltpu.async_copy(table_hbm.at[idx_v], rows_v, sem).wait()

            def sent(s, carry2):
                base = s * SEQ
                acc0 = rows_v[base, pl.ds(0, L)]
                acc1 = rows_v[base, pl.ds(L, L)]
                for r in range(1, SEQ):
                    acc0 = acc0 + rows_v[base + r, pl.ds(0, L)]
                    acc1 = acc1 + rows_v[base + r, pl.ds(L, L)]
                out_v[s, pl.ds(0, L)] = acc0 * (1.0 / SEQ)
                out_v[s, pl.ds(L, L)] = acc1 * (1.0 / SEQ)
                return carry2

            lax.fori_loop(0, CHS, sent, 0)
            pltpu.sync_copy(out_v, out_hbm.at[pl.ds(wid * SENT_PER_W + c * CHS, CHS)])
            return carry

        lax.fori_loop(0, NCHUNK, chunk, 0)


def kernel(token_ids, table):
    ids = token_ids.astype(jnp.int32).reshape(B * SEQ)
    mesh = plsc.VectorSubcoreMesh(
        core_axis_name="c", subcore_axis_name="s", num_cores=NC, num_subcores=NS
    )
    f = pl.kernel(
        _body,
        out_type=jax.ShapeDtypeStruct((B, D), jnp.float32),
        mesh=mesh,
        scratch_types=[
            pltpu.VMEM((TOK_PER_CHUNK,), jnp.int32),
            pltpu.VMEM((TOK_PER_CHUNK, D), jnp.float32),
            pltpu.VMEM((CHS, D), jnp.float32),
            pltpu.SemaphoreType.DMA,
        ],
        compiler_params=pltpu.CompilerParams(use_tc_tiling_on_sc=False),
    )
    return f(ids, table)


# speedup vs baseline: 1.1720x; 1.1720x over previous
import jax
import jax.numpy as jnp
from jax import lax
from jax.experimental import pallas as pl
from jax.experimental.pallas import tpu as pltpu
from jax.experimental.pallas import tpu_sc as plsc

D = 32
SEQ = 50
B = 16384
L = 16
NC, NS = 2, 16
NW = NC * NS
SENT_PER_W = B // NW
CHS = 32
NCHUNK = SENT_PER_W // CHS
TOK = CHS * SEQ


def _body(ids_hbm, table_hbm, out_hbm, idx0, idx1, rows0, rows1, out_v,
          sem0, sem1):
    wid = lax.axis_index("s") * NC + lax.axis_index("c")
    tok_base = wid * (SENT_PER_W * SEQ)
    sent_base = wid * SENT_PER_W

    def fire(c, idx_v, rows_v, sem):
        pltpu.sync_copy(ids_hbm.at[pl.ds(tok_base + c * TOK, TOK)], idx_v)
        pltpu.async_copy(table_hbm.at[idx_v], rows_v, sem)

    def drain_and_pool(c, idx_v, rows_v, sem):
        pltpu.make_async_copy(table_hbm.at[idx_v], rows_v, sem).wait()

        def sent(s, carry):
            base = s * SEQ
            acc0 = rows_v[base, pl.ds(0, L)]
            acc1 = rows_v[base, pl.ds(L, L)]
            for r in range(1, SEQ):
                acc0 = acc0 + rows_v[base + r, pl.ds(0, L)]
                acc1 = acc1 + rows_v[base + r, pl.ds(L, L)]
            out_v[s, pl.ds(0, L)] = acc0 * (1.0 / SEQ)
            out_v[s, pl.ds(L, L)] = acc1 * (1.0 / SEQ)
            return carry

        lax.fori_loop(0, CHS, sent, 0)
        pltpu.sync_copy(out_v, out_hbm.at[pl.ds(sent_base + c * CHS, CHS)])

    fire(0, idx0, rows0, sem0)

    def pair(i, carry):
        a = 2 * i
        b = a + 1
        fire(b, idx1, rows1, sem1)
        drain_and_pool(a, idx0, rows0, sem0)

        @pl.when(b + 1 < NCHUNK)
        def _():
            fire(b + 1, idx0, rows0, sem0)

        drain_and_pool(b, idx1, rows1, sem1)
        return carry

    lax.fori_loop(0, NCHUNK // 2, pair, 0)


def kernel(token_ids, table):
    ids = token_ids.astype(jnp.int32).reshape(B * SEQ)
    mesh = plsc.VectorSubcoreMesh(
        core_axis_name="c", subcore_axis_name="s", num_cores=NC, num_subcores=NS
    )
    f = pl.kernel(
        _body,
        out_type=jax.ShapeDtypeStruct((B, D), jnp.float32),
        mesh=mesh,
        scratch_types=[
            pltpu.VMEM((TOK,), jnp.int32),
            pltpu.VMEM((TOK,), jnp.int32),
            pltpu.VMEM((TOK, D), jnp.float32),
            pltpu.VMEM((TOK, D), jnp.float32),
            pltpu.VMEM((CHS, D), jnp.float32),
            pltpu.SemaphoreType.DMA,
            pltpu.SemaphoreType.DMA,
        ],
        compiler_params=pltpu.CompilerParams(use_tc_tiling_on_sc=False),
    )
    return f(ids, table)
